# baseline (speedup 1.0000x reference)
import jax
import jax.numpy as jnp
from jax.experimental import pallas as pl
from jax.experimental.pallas import tpu as pltpu

_N = 10000
_F = 128
_BM = 200
_H = _N // 2


def _gdn_kernel(x_ref, w_ref, b_ref, adja_ref, adjb_ref, dn_ref, out_ref,
                sup_ref):
    i = pl.program_id(0)

    @pl.when(i <= 1)
    def _support_half():
        off = jnp.minimum(i, 1) * _H
        sup_ref[pl.ds(off, _H), :] = jnp.dot(
            x_ref[...], w_ref[...], preferred_element_type=jnp.float32
        ) + b_ref[...]

    @pl.when(i > 0)
    def _aggregate():
        k = i - 1
        sup_rows = sup_ref[pl.ds(k * _BM, _BM), :]
        dnb = dn_ref[...]

        @pl.when(k % 2 == 0)
        def _even():
            neigh = jnp.dot(adja_ref[...], sup_ref[...],
                            preferred_element_type=jnp.float32)
            out_ref[...] = jnp.maximum(sup_rows - dnb * neigh, 0.0)

        @pl.when(k % 2 == 1)
        def _odd():
            neigh = jnp.dot(adjb_ref[...], sup_ref[...],
                            preferred_element_type=jnp.float32)
            out_ref[...] = jnp.maximum(sup_rows - dnb * neigh, 0.0)


def kernel(x, adj_matrix, degree_norm, W, b):
    b2 = b.reshape(1, _F)
    num_i = _N // _BM

    def _clamped(i):
        return (jnp.maximum(i - 1, 0), 0)

    def _even_blocks(i):
        return (jnp.minimum(2 * (i // 2), num_i - 2), 0)

    def _odd_blocks(i):
        return (2 * jnp.maximum((i - 1) // 2, 0) + 1, 0)

    out = pl.pallas_call(
        _gdn_kernel,
        grid=(num_i + 1,),
        in_specs=[
            pl.BlockSpec((_H, _F), lambda i: (jnp.minimum(i, 1), 0)),
            pl.BlockSpec((_F, _F), lambda i: (0, 0)),
            pl.BlockSpec((1, _F), lambda i: (0, 0)),
            pl.BlockSpec((_BM, _N), _even_blocks),
            pl.BlockSpec((_BM, _N), _odd_blocks),
            pl.BlockSpec((_BM, 1), _clamped),
        ],
        out_specs=pl.BlockSpec((_BM, _F), _clamped),
        out_shape=jax.ShapeDtypeStruct((_N, _F), jnp.float32),
        scratch_shapes=[pltpu.VMEM((_N, _F), jnp.float32)],
        compiler_params=pltpu.CompilerParams(
            dimension_semantics=("arbitrary",)),
    )(x, W, b2, adj_matrix, adj_matrix, degree_norm)
    return out


# speedup vs baseline: 1.0238x; 1.0238x over previous
import jax
import jax.numpy as jnp
from jax.experimental import pallas as pl
from jax.experimental.pallas import tpu as pltpu

_N = 10000
_F = 128
_BM = 200
_H = _N // 2


def _gdn_kernel(x_ref, w_ref, b_ref, adj_ref, dn_ref, out_ref, sup_ref):
    i = pl.program_id(0)

    @pl.when(i <= 1)
    def _support_half():
        off = jnp.minimum(i, 1) * _H
        sup_ref[pl.ds(off, _H), :] = jnp.dot(
            x_ref[...], w_ref[...], preferred_element_type=jnp.float32
        ) + b_ref[...]

    @pl.when(i > 0)
    def _aggregate():
        neigh = jnp.dot(adj_ref[...], sup_ref[...],
                        preferred_element_type=jnp.float32)
        sup_rows = sup_ref[pl.ds((i - 1) * _BM, _BM), :]
        out_ref[...] = jnp.maximum(sup_rows - dn_ref[...] * neigh, 0.0)


def kernel(x, adj_matrix, degree_norm, W, b):
    b2 = b.reshape(1, _F)
    num_i = _N // _BM

    def _clamped(i):
        return (jnp.maximum(i - 1, 0), 0)

    out = pl.pallas_call(
        _gdn_kernel,
        grid=(num_i + 1,),
        in_specs=[
            pl.BlockSpec((_H, _F), lambda i: (jnp.minimum(i, 1), 0)),
            pl.BlockSpec((_F, _F), lambda i: (0, 0)),
            pl.BlockSpec((1, _F), lambda i: (0, 0)),
            pl.BlockSpec((_BM, _N), _clamped),
            pl.BlockSpec((_BM, 1), _clamped),
        ],
        out_specs=pl.BlockSpec((_BM, _F), _clamped),
        out_shape=jax.ShapeDtypeStruct((_N, _F), jnp.float32),
        scratch_shapes=[pltpu.VMEM((_N, _F), jnp.float32)],
        compiler_params=pltpu.CompilerParams(
            dimension_semantics=("arbitrary",)),
    )(x, W, b2, adj_matrix, degree_norm)
    return out
